# Initial kernel scaffold; baseline (speedup 1.0000x reference)
import jax
import jax.numpy as jnp
from jax.experimental import pallas as pl


def kernel(predictions, targets):
    raise NotImplementedError("write your pallas kernel here")


# speedup vs baseline: 1.4330x; 1.4330x over previous
import jax
import jax.numpy as jnp
from jax.experimental import pallas as pl
from jax.experimental.pallas import tpu as pltpu

_LAMBDA_COORD = 5.0
_LAMBDA_NOOBJ = 0.5
_C = 80
_G = 96
_S = 4096
_B = 32


def _body(pred_ref, targ_ref, out_ref):
    p = pred_ref[0]
    t = targ_ref[0]
    f32 = jnp.float32

    p0 = p[0:1, :]
    p1 = p[1:2, :]
    p2 = p[2:3, :]
    p3 = p[3:4, :]
    z = p[4:5, :]
    cs = p[5:, :]

    sx = 1.0 / (1.0 + jnp.exp(-p0))
    sy = 1.0 / (1.0 + jnp.exp(-p1))
    g = jnp.maximum(z, 0.0) + jnp.log(1.0 + jnp.exp(-jnp.abs(z)))
    mx = jnp.max(cs, axis=0, keepdims=True)
    lse = mx + jnp.log(jnp.sum(jnp.exp(cs - mx), axis=0, keepdims=True))
    ones = jnp.ones_like(p0)

    P = jnp.concatenate(
        [sx, sx * sx, sy, sy * sy, p2, p2 * p2, p3, p3 * p3, z, g, lse, ones],
        axis=0)

    t0 = t[:, 0:1]
    t1 = t[:, 1:2]
    t2 = t[:, 2:3]
    t3 = t[:, 3:4]
    t4 = t[:, 4:5]
    tc = t[:, 5:]

    o = (t4 == 1.0).astype(f32)
    nb = (t4 == 0.0).astype(f32)
    q = t0 * t0 + t1 * t1 + t2 * t2 + t3 * t3
    T1 = jnp.concatenate(
        [o, o * t0, o * t1, o * t2, o * t3, nb, o * q], axis=1)

    dn = (((1,), (0,)), ((), ()))
    M1 = jax.lax.dot_general(P, T1, dn, precision=jax.lax.Precision.HIGHEST,
                             preferred_element_type=f32)

    maxv = jnp.max(tc, axis=1, keepdims=True)
    iota_c = jax.lax.broadcasted_iota(jnp.int32, tc.shape, 1)
    cand = jnp.where(tc == maxv, iota_c, _C)
    idxm = jnp.min(cand, axis=1, keepdims=True)
    W = jnp.where(cand == idxm, o, 0.0)

    M2 = jax.lax.dot_general(cs, W, dn, precision=jax.lax.Precision.HIGHEST,
                             preferred_element_type=f32)
    di = jax.lax.broadcasted_iota(jnp.int32, (_C, _C), 0)
    dj = jax.lax.broadcasted_iota(jnp.int32, (_C, _C), 1)
    picked = jnp.sum(jnp.where(di == dj, M2, 0.0), keepdims=True)

    loc = (M1[1:2, 0:1] + M1[3:4, 0:1] + M1[5:6, 0:1] + M1[7:8, 0:1]
           - 2.0 * (M1[0:1, 1:2] + M1[2:3, 2:3] + M1[4:5, 3:4] + M1[6:7, 4:5])
           + M1[11:12, 6:7]) * _LAMBDA_COORD
    conf = (M1[9:10, 0:1] - M1[8:9, 0:1]) + _LAMBDA_NOOBJ * M1[9:10, 5:6]
    cls = M1[10:11, 0:1] - picked

    partial = (loc + conf + cls) * (1.0 / _B)

    @pl.when(pl.program_id(0) == 0)
    def _():
        out_ref[...] = jnp.zeros_like(out_ref)

    out_ref[...] += partial


def kernel(predictions, targets):
    pred_r = predictions.reshape(_G, 85, _S)
    targ_r = targets.reshape(_G, _S, 85)
    out = pl.pallas_call(
        _body,
        grid=(_G,),
        in_specs=[
            pl.BlockSpec((1, 85, _S), lambda i: (i, 0, 0)),
            pl.BlockSpec((1, _S, 85), lambda i: (i, 0, 0)),
        ],
        out_specs=pl.BlockSpec((1, 1), lambda i: (0, 0)),
        out_shape=jax.ShapeDtypeStruct((1, 1), jnp.float32),
        compiler_params=pltpu.CompilerParams(
            dimension_semantics=("arbitrary",)),
    )(pred_r, targ_r)
    return out[0, 0]


# speedup vs baseline: 4.8591x; 3.3909x over previous
import jax
import jax.numpy as jnp
from jax.experimental import pallas as pl
from jax.experimental.pallas import tpu as pltpu

_LAMBDA_COORD = 5.0
_LAMBDA_NOOBJ = 0.5
_C = 80
_S = 4096
_B = 32
_A = 3


def _body(pred_ref, targ_ref, out_ref):
    f32 = jnp.float32
    p = pred_ref[0]
    t2 = targ_ref[0, 0].reshape(_S, 85)
    t = jnp.swapaxes(t2, 0, 1)

    p0 = p[0:1, :]
    p1 = p[1:2, :]
    p2 = p[2:3, :]
    p3 = p[3:4, :]
    z = p[4:5, :]
    cs = p[5:, :]

    t0 = t[0:1, :]
    t1 = t[1:2, :]
    t2r = t[2:3, :]
    t3 = t[3:4, :]
    t4 = t[4:5, :]
    tc = t[5:, :]

    o = (t4 == 1.0).astype(f32)
    nb = (t4 == 0.0).astype(f32)

    sx = 1.0 / (1.0 + jnp.exp(-p0))
    sy = 1.0 / (1.0 + jnp.exp(-p1))
    dx = sx - t0
    dy = sy - t1
    dw = p2 - t2r
    dh = p3 - t3
    loc_row = o * (dx * dx + dy * dy + dw * dw + dh * dh)

    g = jnp.maximum(z, 0.0) + jnp.log(1.0 + jnp.exp(-jnp.abs(z)))
    conf_row = o * (g - z) + _LAMBDA_NOOBJ * (nb * g)

    mx = jnp.max(cs, axis=0, keepdims=True)
    lse = mx + jnp.log(jnp.sum(jnp.exp(cs - mx), axis=0, keepdims=True))

    maxv = jnp.max(tc, axis=0, keepdims=True)
    iota_c = jax.lax.broadcasted_iota(jnp.int32, tc.shape, 0)
    cand = jnp.where(tc == maxv, iota_c, _C)
    idxm = jnp.min(cand, axis=0, keepdims=True)
    picked = jnp.sum(jnp.where(cand == idxm, cs, 0.0), axis=0,
                     keepdims=True)
    cls_row = o * (lse - picked)

    total_row = _LAMBDA_COORD * loc_row + conf_row + cls_row
    partial = jnp.sum(total_row, axis=1, keepdims=True) * (1.0 / _B)

    @pl.when(jnp.logical_and(pl.program_id(0) == 0, pl.program_id(1) == 0))
    def _():
        out_ref[...] = jnp.zeros_like(out_ref)

    out_ref[...] += partial


def kernel(predictions, targets):
    pred_r = predictions.reshape(_B * _A, 85, _S)
    out = pl.pallas_call(
        _body,
        grid=(_B, _A),
        in_specs=[
            pl.BlockSpec((1, 85, _S), lambda b, a: (b * _A + a, 0, 0)),
            pl.BlockSpec((1, 1, 64, 64, 85), lambda b, a: (b, a, 0, 0, 0)),
        ],
        out_specs=pl.BlockSpec((1, 1), lambda b, a: (0, 0)),
        out_shape=jax.ShapeDtypeStruct((1, 1), jnp.float32),
        compiler_params=pltpu.CompilerParams(
            dimension_semantics=("arbitrary", "arbitrary")),
    )(pred_r, targets)
    return out[0, 0]


# speedup vs baseline: 8.5950x; 1.7688x over previous
import jax
import jax.numpy as jnp
from jax.experimental import pallas as pl
from jax.experimental.pallas import tpu as pltpu

_LAMBDA_COORD = 5.0
_LAMBDA_NOOBJ = 0.5
_C = 80
_S = 4096
_B = 32
_A = 3


def _group_loss(p, t):
    f32 = jnp.float32
    p0 = p[0:1, :]
    p1 = p[1:2, :]
    p2 = p[2:3, :]
    p3 = p[3:4, :]
    z = p[4:5, :]
    cs = p[5:, :]

    t0 = t[0:1, :]
    t1 = t[1:2, :]
    t2r = t[2:3, :]
    t3 = t[3:4, :]
    t4 = t[4:5, :]
    tc = t[5:, :]

    o = (t4 == 1.0).astype(f32)
    nb = (t4 == 0.0).astype(f32)

    sx = 1.0 / (1.0 + jnp.exp(-p0))
    sy = 1.0 / (1.0 + jnp.exp(-p1))
    dx = sx - t0
    dy = sy - t1
    dw = p2 - t2r
    dh = p3 - t3
    loc_row = o * (dx * dx + dy * dy + dw * dw + dh * dh)

    g = jnp.maximum(z, 0.0) + jnp.log(1.0 + jnp.exp(-jnp.abs(z)))
    conf_row = o * (g - z) + _LAMBDA_NOOBJ * (nb * g)

    mx = jnp.max(cs, axis=0, keepdims=True)
    lse = mx + jnp.log(jnp.sum(jnp.exp(cs - mx), axis=0, keepdims=True))

    maxv = jnp.max(tc, axis=0, keepdims=True)
    iota_c = jax.lax.broadcasted_iota(jnp.int32, tc.shape, 0)
    cand = jnp.where(tc == maxv, iota_c, _C)
    idxm = jnp.min(cand, axis=0, keepdims=True)
    picked = jnp.sum(jnp.where(cand == idxm, cs, 0.0), axis=0,
                     keepdims=True)
    cls_row = o * (lse - picked)

    total_row = _LAMBDA_COORD * loc_row + conf_row + cls_row
    return jnp.sum(total_row, axis=1, keepdims=True)


def _body(pred_ref, targ_ref, out_ref):
    acc = jnp.zeros((1, 1), jnp.float32)
    for a in range(_A):
        p = pred_ref[0, 85 * a:85 * (a + 1), :]
        t = jnp.swapaxes(targ_ref[0, a].reshape(_S, 85), 0, 1)
        acc = acc + _group_loss(p, t)

    @pl.when(pl.program_id(0) == 0)
    def _():
        out_ref[...] = jnp.zeros_like(out_ref)

    out_ref[...] += acc * (1.0 / _B)


def kernel(predictions, targets):
    pred_r = predictions.reshape(_B, _A * 85, _S)
    out = pl.pallas_call(
        _body,
        grid=(_B,),
        in_specs=[
            pl.BlockSpec((1, _A * 85, _S), lambda b: (b, 0, 0)),
            pl.BlockSpec((1, _A, 64, 64, 85), lambda b: (b, 0, 0, 0, 0)),
        ],
        out_specs=pl.BlockSpec((1, 1), lambda b: (0, 0)),
        out_shape=jax.ShapeDtypeStruct((1, 1), jnp.float32),
        compiler_params=pltpu.CompilerParams(
            dimension_semantics=("arbitrary",)),
    )(pred_r, targets)
    return out[0, 0]


# speedup vs baseline: 8.7552x; 1.0186x over previous
import jax
import jax.numpy as jnp
from jax.experimental import pallas as pl
from jax.experimental.pallas import tpu as pltpu

_LAMBDA_COORD = 5.0
_LAMBDA_NOOBJ = 0.5
_C = 80
_S = 4096
_B = 32
_A = 3


def _group_loss(p, t):
    f32 = jnp.float32
    p0 = p[0:1, :]
    p1 = p[1:2, :]
    p2 = p[2:3, :]
    p3 = p[3:4, :]
    z = p[4:5, :]
    cs = p[5:, :]

    t0 = t[0:1, :]
    t1 = t[1:2, :]
    t2r = t[2:3, :]
    t3 = t[3:4, :]
    t4 = t[4:5, :]
    tc = t[5:, :]

    o = (t4 == 1.0).astype(f32)
    nb = (t4 == 0.0).astype(f32)

    sx = 1.0 / (1.0 + jnp.exp(-p0))
    sy = 1.0 / (1.0 + jnp.exp(-p1))
    dx = sx - t0
    dy = sy - t1
    dw = p2 - t2r
    dh = p3 - t3
    loc_row = o * (dx * dx + dy * dy + dw * dw + dh * dh)

    g = jnp.maximum(z, 0.0) + jnp.log(1.0 + jnp.exp(-jnp.abs(z)))
    conf_row = o * (g - z) + _LAMBDA_NOOBJ * (nb * g)

    mx = jnp.max(cs, axis=0, keepdims=True)
    lse = mx + jnp.log(jnp.sum(jnp.exp(cs - mx), axis=0, keepdims=True))

    maxv = jnp.max(tc, axis=0, keepdims=True)
    iota_c = jax.lax.broadcasted_iota(jnp.int32, tc.shape, 0)
    cand = jnp.where(tc == maxv, iota_c, _C)
    idxm = jnp.min(cand, axis=0, keepdims=True)
    picked = jnp.sum(jnp.where(cand == idxm, cs, 0.0), axis=0,
                     keepdims=True)
    cls_row = o * (lse - picked)

    total_row = _LAMBDA_COORD * loc_row + conf_row + cls_row
    return jnp.sum(total_row, axis=1, keepdims=True)


_BB = 2


def _body(pred_ref, targ_ref, out_ref):
    acc = jnp.zeros((1, 1), jnp.float32)
    for b in range(_BB):
        for a in range(_A):
            p = pred_ref[b, 85 * a:85 * (a + 1), :]
            t = jnp.swapaxes(targ_ref[b, a].reshape(_S, 85), 0, 1)
            acc = acc + _group_loss(p, t)

    @pl.when(pl.program_id(0) == 0)
    def _():
        out_ref[...] = jnp.zeros_like(out_ref)

    out_ref[...] += acc * (1.0 / _B)


def kernel(predictions, targets):
    pred_r = predictions.reshape(_B, _A * 85, _S)
    out = pl.pallas_call(
        _body,
        grid=(_B // _BB,),
        in_specs=[
            pl.BlockSpec((_BB, _A * 85, _S), lambda b: (b, 0, 0)),
            pl.BlockSpec((_BB, _A, 64, 64, 85), lambda b: (b, 0, 0, 0, 0)),
        ],
        out_specs=pl.BlockSpec((1, 1), lambda b: (0, 0)),
        out_shape=jax.ShapeDtypeStruct((1, 1), jnp.float32),
        compiler_params=pltpu.CompilerParams(
            dimension_semantics=("arbitrary",)),
    )(pred_r, targets)
    return out[0, 0]
